# Initial kernel scaffold; baseline (speedup 1.0000x reference)
import jax
import jax.numpy as jnp
from jax.experimental import pallas as pl


def kernel(logits):
    raise NotImplementedError("write your pallas kernel here")


# speedup vs baseline: 20.1754x; 20.1754x over previous
import functools
import math

import jax
import jax.numpy as jnp
from jax.experimental import pallas as pl
from jax.experimental.pallas import tpu as pltpu

_T = 1.3
_QHAT = 0.9
_KREG = 5
_LAMDA = 0.01
_C = 512
_NBISECT = 8


def _select_body(x_ref, sizes_ref, vkey_ref, rallow_ref, mrow_ref,
                 m_s, s_s, sinv_s, lo_s, cnt_s, sum_s):
    p = pl.program_id(0)
    b = pl.program_id(1)
    nb = pl.num_programs(1)
    y = x_ref[...] / _T

    @pl.when(p == 0)
    def _():
        @pl.when(b == 0)
        def _():
            m_s[...] = jnp.full(m_s.shape, -jnp.inf, jnp.float32)
        m_s[...] = jnp.maximum(m_s[...], jnp.max(y, axis=1, keepdims=True))

    @pl.when(p == 1)
    def _():
        @pl.when(b == 0)
        def _():
            s_s[...] = jnp.zeros(s_s.shape, jnp.float32)
        e = jnp.exp(y - m_s[...])
        s_s[...] = s_s[...] + jnp.sum(e, axis=1, keepdims=True)

        @pl.when(b == nb - 1)
        def _():
            sinv_s[...] = 1.0 / s_s[...]

    @pl.when((p >= 2) & (p <= 1 + _NBISECT))
    def _():
        @pl.when(b == 0)
        def _():
            cnt_s[...] = jnp.zeros(cnt_s.shape, jnp.float32)
            sum_s[...] = jnp.zeros(sum_s.shape, jnp.float32)

            @pl.when(p == 2)
            def _():
                lo_s[...] = jnp.zeros(lo_s.shape, jnp.int32)

        e = jnp.exp(y - m_s[...])
        key = jax.lax.bitcast_convert_type(e, jnp.int32)
        shift = 34 - 4 * p
        shift = jnp.maximum(shift, 0)
        step = jnp.left_shift(jnp.int32(1), shift)
        lo = lo_s[...]
        for j in range(1, 16):
            tj = lo + j * step
            msk = key >= tj
            cnt_s[:, j:j + 1] += jnp.sum(jnp.where(msk, 1.0, 0.0), axis=1,
                                         keepdims=True)
            sum_s[:, j:j + 1] += jnp.sum(jnp.where(msk, e, 0.0), axis=1,
                                         keepdims=True)

        @pl.when(b == nb - 1)
        def _():
            cnt = cnt_s[...]
            fval = sum_s[...] * sinv_s[...] + _LAMDA * jnp.maximum(
                cnt - _KREG, 0.0)
            colj = jax.lax.broadcasted_iota(jnp.int32, fval.shape, 1)
            pred = jnp.logical_or(fval > _QHAT, colj == 0)
            jstar = (jnp.sum(jnp.where(pred, 1.0, 0.0), axis=1,
                             keepdims=True) - 1.0).astype(jnp.int32)
            lo_s[...] = lo + jstar * step

    @pl.when(p == 2 + _NBISECT)
    def _():
        @pl.when(b == 0)
        def _():
            cnt_s[...] = jnp.zeros(cnt_s.shape, jnp.float32)
            sum_s[...] = jnp.zeros(sum_s.shape, jnp.float32)

        e = jnp.exp(y - m_s[...])
        key = jax.lax.bitcast_convert_type(e, jnp.int32)
        vk = lo_s[...]
        gt = key > vk
        eq = key == vk
        cnt_s[:, 0:1] += jnp.sum(jnp.where(gt, 1.0, 0.0), axis=1,
                                 keepdims=True)
        cnt_s[:, 1:2] += jnp.sum(jnp.where(eq, 1.0, 0.0), axis=1,
                                 keepdims=True)
        sum_s[:, 0:1] += jnp.sum(jnp.where(gt, e, 0.0), axis=1,
                                 keepdims=True)

        @pl.when(b == nb - 1)
        def _():
            sinv = sinv_s[...]
            cnt_gt = cnt_s[:, 0:1]
            cnt_eq = cnt_s[:, 1:2]
            sum_gt = sum_s[:, 0:1] * sinv
            sv = jax.lax.bitcast_convert_type(vk, jnp.float32) * sinv
            bsz = vk.shape[0]
            jj = jax.lax.broadcasted_iota(
                jnp.int32, (bsz, 128), 1).astype(jnp.float32) + 1.0
            mpos = cnt_gt + jj
            fj = sum_gt + jj * sv + _LAMDA * jnp.maximum(mpos - _KREG, 0.0)
            hold = jnp.logical_and(jj <= cnt_eq, fj <= _QHAT)
            qc = jnp.sum(jnp.where(hold, 1.0, 0.0), axis=1, keepdims=True)
            sizes_ref[...] = (cnt_gt + qc + 1.0).astype(jnp.int32)
            rallow_ref[...] = (qc + 1.0).astype(jnp.int32)
            vkey_ref[...] = vk
            mrow_ref[...] = m_s[...]


def _mask_body(x_ref, mrow_ref, vkey_ref, rallow_ref, out_ref, carry_s, lt_s):
    b = pl.program_id(0)

    @pl.when(b == 0)
    def _():
        carry_s[...] = jnp.zeros(carry_s.shape, jnp.float32)
        r = jax.lax.broadcasted_iota(jnp.int32, lt_s.shape, 0)
        c = jax.lax.broadcasted_iota(jnp.int32, lt_s.shape, 1)
        lt_s[...] = jnp.where(r < c, 1.0, 0.0)

    y = x_ref[...] / _T
    e = jnp.exp(y - mrow_ref[...])
    key = jax.lax.bitcast_convert_type(e, jnp.int32)
    vk = vkey_ref[...]
    gt = key > vk
    eq = key == vk
    eqf = jnp.where(eq, 1.0, 0.0)
    rank = carry_s[...] + jnp.dot(eqf, lt_s[...],
                                  preferred_element_type=jnp.float32)
    ra = rallow_ref[...].astype(jnp.float32)
    out_ref[...] = jnp.logical_or(gt, jnp.logical_and(eq, rank < ra))
    carry_s[...] = carry_s[...] + jnp.sum(eqf, axis=1, keepdims=True)


@functools.partial(jax.jit, static_argnames=("interpret",))
def kernel(logits, interpret=False):
    bsz, n = logits.shape
    nb = math.ceil(n / _C)
    npad = nb * _C
    xp = logits
    if npad != n:
        xp = jnp.pad(logits, ((0, 0), (0, npad - n)),
                     constant_values=-jnp.inf)

    npasses = 3 + _NBISECT
    row = functools.partial(pl.BlockSpec, (bsz, 1))
    sizes, vkey, rallow, mrow = pl.pallas_call(
        _select_body,
        grid=(npasses, nb),
        in_specs=[pl.BlockSpec((bsz, _C), lambda p, b: (0, b))],
        out_specs=[row(lambda p, b: (0, 0)) for _ in range(4)],
        out_shape=[
            jax.ShapeDtypeStruct((bsz, 1), jnp.int32),
            jax.ShapeDtypeStruct((bsz, 1), jnp.int32),
            jax.ShapeDtypeStruct((bsz, 1), jnp.int32),
            jax.ShapeDtypeStruct((bsz, 1), jnp.float32),
        ],
        scratch_shapes=[
            pltpu.VMEM((bsz, 1), jnp.float32),
            pltpu.VMEM((bsz, 1), jnp.float32),
            pltpu.VMEM((bsz, 1), jnp.float32),
            pltpu.VMEM((bsz, 1), jnp.int32),
            pltpu.VMEM((bsz, 16), jnp.float32),
            pltpu.VMEM((bsz, 16), jnp.float32),
        ],
        interpret=interpret,
    )(xp)

    mask = pl.pallas_call(
        _mask_body,
        grid=(nb,),
        in_specs=[
            pl.BlockSpec((bsz, _C), lambda b: (0, b)),
            pl.BlockSpec((bsz, 1), lambda b: (0, 0)),
            pl.BlockSpec((bsz, 1), lambda b: (0, 0)),
            pl.BlockSpec((bsz, 1), lambda b: (0, 0)),
        ],
        out_specs=pl.BlockSpec((bsz, _C), lambda b: (0, b)),
        out_shape=jax.ShapeDtypeStruct((bsz, npad), jnp.bool_),
        scratch_shapes=[
            pltpu.VMEM((bsz, 1), jnp.float32),
            pltpu.VMEM((_C, _C), jnp.float32),
        ],
        interpret=interpret,
    )(xp, mrow, vkey, rallow)

    return (logits, mask[:, :n], sizes.reshape(bsz))


# speedup vs baseline: 36.9418x; 1.8310x over previous
import functools
import math

import jax
import jax.numpy as jnp
from jax.experimental import pallas as pl
from jax.experimental.pallas import tpu as pltpu

_T = 1.3
_QHAT = 0.9
_KREG = 5
_LAMDA = 0.01
_CA = 2048
_CB = 512
_NBISECT = 8


def _select_body(x_ref, sizes_ref, vkey_ref, rallow_ref, mrow_ref,
                 m_s, s_s, sinv_s, lo_s, cnt_s, sum_s):
    p = pl.program_id(0)
    b = pl.program_id(1)
    nb = pl.num_programs(1)
    y = x_ref[...] / _T

    @pl.when(p == 0)
    def _():
        @pl.when(b == 0)
        def _():
            m_s[...] = jnp.full(m_s.shape, -jnp.inf, jnp.float32)
            s_s[...] = jnp.zeros(s_s.shape, jnp.float32)
        m_old = m_s[...]
        m_new = jnp.maximum(m_old, jnp.max(y, axis=1, keepdims=True))
        e = jnp.exp(y - m_new)
        s_s[...] = s_s[...] * jnp.exp(m_old - m_new) + jnp.sum(
            e, axis=1, keepdims=True)
        m_s[...] = m_new

        @pl.when(b == nb - 1)
        def _():
            sinv_s[...] = 1.0 / s_s[...]

    @pl.when((p >= 1) & (p <= _NBISECT))
    def _():
        @pl.when(b == 0)
        def _():
            cnt_s[...] = jnp.zeros(cnt_s.shape, jnp.float32)
            sum_s[...] = jnp.zeros(sum_s.shape, jnp.float32)

            @pl.when(p == 1)
            def _():
                lo_s[...] = jnp.zeros(lo_s.shape, jnp.int32)

        e = jnp.exp(y - m_s[...])
        key = jax.lax.bitcast_convert_type(e, jnp.int32)
        shift = 30 - 4 * p
        shift = jnp.maximum(shift, 0)
        step = jnp.left_shift(jnp.int32(1), shift)
        lo = lo_s[...]
        cparts = []
        sparts = []
        for j in range(1, 16):
            msk = key >= lo + j * step
            cparts.append(jnp.sum(jnp.where(msk, 1.0, 0.0), axis=1,
                                  keepdims=True))
            sparts.append(jnp.sum(jnp.where(msk, e, 0.0), axis=1,
                                  keepdims=True))
        cnt_s[:, 1:16] += jnp.concatenate(cparts, axis=1)
        sum_s[:, 1:16] += jnp.concatenate(sparts, axis=1)

        @pl.when(b == nb - 1)
        def _():
            cnt = cnt_s[...]
            fval = sum_s[...] * sinv_s[...] + _LAMDA * jnp.maximum(
                cnt - _KREG, 0.0)
            colj = jax.lax.broadcasted_iota(jnp.int32, fval.shape, 1)
            pred = jnp.logical_or(fval > _QHAT, colj == 0)
            jstar = (jnp.sum(jnp.where(pred, 1.0, 0.0), axis=1,
                             keepdims=True) - 1.0).astype(jnp.int32)
            lo_s[...] = lo + jstar * step

    @pl.when(p == 1 + _NBISECT)
    def _():
        @pl.when(b == 0)
        def _():
            cnt_s[...] = jnp.zeros(cnt_s.shape, jnp.float32)
            sum_s[...] = jnp.zeros(sum_s.shape, jnp.float32)

        e = jnp.exp(y - m_s[...])
        key = jax.lax.bitcast_convert_type(e, jnp.int32)
        vk = lo_s[...]
        gt = key > vk
        eq = key == vk
        cnt_s[:, 0:1] += jnp.sum(jnp.where(gt, 1.0, 0.0), axis=1,
                                 keepdims=True)
        cnt_s[:, 1:2] += jnp.sum(jnp.where(eq, 1.0, 0.0), axis=1,
                                 keepdims=True)
        sum_s[:, 0:1] += jnp.sum(jnp.where(gt, e, 0.0), axis=1,
                                 keepdims=True)

        @pl.when(b == nb - 1)
        def _():
            sinv = sinv_s[...]
            cnt_gt = cnt_s[:, 0:1]
            cnt_eq = cnt_s[:, 1:2]
            sum_gt = sum_s[:, 0:1] * sinv
            sv = jax.lax.bitcast_convert_type(vk, jnp.float32) * sinv
            bsz = vk.shape[0]
            jj = jax.lax.broadcasted_iota(
                jnp.int32, (bsz, 128), 1).astype(jnp.float32) + 1.0
            mpos = cnt_gt + jj
            fj = sum_gt + jj * sv + _LAMDA * jnp.maximum(mpos - _KREG, 0.0)
            hold = jnp.logical_and(jj <= cnt_eq, fj <= _QHAT)
            qc = jnp.sum(jnp.where(hold, 1.0, 0.0), axis=1, keepdims=True)
            sizes_ref[...] = (cnt_gt + qc + 1.0).astype(jnp.int32)
            rallow_ref[...] = (qc + 1.0).astype(jnp.int32)
            vkey_ref[...] = vk
            mrow_ref[...] = m_s[...]


def _mask_body(x_ref, mrow_ref, vkey_ref, rallow_ref, out_ref, carry_s, lt_s):
    b = pl.program_id(0)

    @pl.when(b == 0)
    def _():
        carry_s[...] = jnp.zeros(carry_s.shape, jnp.float32)
        r = jax.lax.broadcasted_iota(jnp.int32, lt_s.shape, 0)
        c = jax.lax.broadcasted_iota(jnp.int32, lt_s.shape, 1)
        lt_s[...] = jnp.where(r < c, 1.0, 0.0)

    y = x_ref[...] / _T
    e = jnp.exp(y - mrow_ref[...])
    key = jax.lax.bitcast_convert_type(e, jnp.int32)
    vk = vkey_ref[...]
    gt = key > vk
    eq = key == vk
    eqf = jnp.where(eq, 1.0, 0.0)
    rank = carry_s[...] + jnp.dot(eqf, lt_s[...],
                                  preferred_element_type=jnp.float32)
    ra = rallow_ref[...].astype(jnp.float32)
    out_ref[...] = jnp.logical_or(gt, jnp.logical_and(eq, rank < ra))
    carry_s[...] = carry_s[...] + jnp.sum(eqf, axis=1, keepdims=True)


@functools.partial(jax.jit, static_argnames=("interpret",))
def kernel(logits, interpret=False):
    bsz, n = logits.shape
    npad = math.lcm(_CA, _CB) * math.ceil(n / math.lcm(_CA, _CB))
    nba = npad // _CA
    nbb = npad // _CB
    xp = logits
    if npad != n:
        xp = jnp.pad(logits, ((0, 0), (0, npad - n)),
                     constant_values=-jnp.inf)

    npasses = 2 + _NBISECT
    row = functools.partial(pl.BlockSpec, (bsz, 1))
    sizes, vkey, rallow, mrow = pl.pallas_call(
        _select_body,
        grid=(npasses, nba),
        in_specs=[pl.BlockSpec((bsz, _CA), lambda p, b: (0, b))],
        out_specs=[row(lambda p, b: (0, 0)) for _ in range(4)],
        out_shape=[
            jax.ShapeDtypeStruct((bsz, 1), jnp.int32),
            jax.ShapeDtypeStruct((bsz, 1), jnp.int32),
            jax.ShapeDtypeStruct((bsz, 1), jnp.int32),
            jax.ShapeDtypeStruct((bsz, 1), jnp.float32),
        ],
        scratch_shapes=[
            pltpu.VMEM((bsz, 1), jnp.float32),
            pltpu.VMEM((bsz, 1), jnp.float32),
            pltpu.VMEM((bsz, 1), jnp.float32),
            pltpu.VMEM((bsz, 1), jnp.int32),
            pltpu.VMEM((bsz, 16), jnp.float32),
            pltpu.VMEM((bsz, 16), jnp.float32),
        ],
        interpret=interpret,
    )(xp)

    mask = pl.pallas_call(
        _mask_body,
        grid=(nbb,),
        in_specs=[
            pl.BlockSpec((bsz, _CB), lambda b: (0, b)),
            pl.BlockSpec((bsz, 1), lambda b: (0, 0)),
            pl.BlockSpec((bsz, 1), lambda b: (0, 0)),
            pl.BlockSpec((bsz, 1), lambda b: (0, 0)),
        ],
        out_specs=pl.BlockSpec((bsz, _CB), lambda b: (0, b)),
        out_shape=jax.ShapeDtypeStruct((bsz, npad), jnp.bool_),
        scratch_shapes=[
            pltpu.VMEM((bsz, 1), jnp.float32),
            pltpu.VMEM((_CB, _CB), jnp.float32),
        ],
        interpret=interpret,
    )(xp, mrow, vkey, rallow)

    return (logits, mask[:, :n], sizes.reshape(bsz))
